# Initial kernel scaffold; baseline (speedup 1.0000x reference)
import jax
import jax.numpy as jnp
from jax.experimental import pallas as pl


def kernel(inputs, targets, global_inputs, global_targets, margin):
    raise NotImplementedError("write your pallas kernel here")


# speedup vs baseline: 1.2204x; 1.2204x over previous
import functools

import jax
import jax.numpy as jnp
from jax.experimental import pallas as pl
from jax.experimental.pallas import tpu as pltpu

N = 1024
M = 16384
D = 512
ALPHA = 10.0
BETA = 2.0
BASE = 0.5

BM = 2048


def _body(x_ref, t_ref, g_ref, gt_ref, o_ref,
          acc_pos, acc_neg, cnt_pos, cnt_neg, *, num_steps):
    j = pl.program_id(0)
    s = jax.lax.dot_general(
        x_ref[...], g_ref[...],
        dimension_numbers=(((1,), (1,)), ((), ())),
        preferred_element_type=jnp.float32,
    )
    same = t_ref[...] == gt_ref[...]
    posm = same & (s < 1.0)
    pos_t = jnp.where(posm, jnp.exp(-BETA * (s - BASE)), 0.0)
    neg_t = jnp.where(same, 0.0, jnp.exp(ALPHA * (s - BASE)))
    ps = jnp.sum(pos_t, axis=1, keepdims=True)
    ns = jnp.sum(neg_t, axis=1, keepdims=True)
    pc = jnp.sum(posm.astype(jnp.float32), axis=1, keepdims=True)
    nc = jnp.sum(jnp.where(same, 0.0, 1.0), axis=1, keepdims=True)

    @pl.when(j == 0)
    def _init():
        acc_pos[...] = ps
        acc_neg[...] = ns
        cnt_pos[...] = pc
        cnt_neg[...] = nc

    @pl.when(j > 0)
    def _accum():
        acc_pos[...] += ps
        acc_neg[...] += ns
        cnt_pos[...] += pc
        cnt_neg[...] += nc

    @pl.when(j == num_steps - 1)
    def _finish():
        pos_loss = (2.0 / BETA) * jnp.log(1.0 + acc_pos[...])
        neg_loss = (2.0 / ALPHA) * jnp.log(1.0 + acc_neg[...])
        valid = (cnt_pos[...] > 0.0) & (cnt_neg[...] > 0.0)
        per = jnp.where(valid, pos_loss + neg_loss, 0.0)
        o_ref[...] = jnp.sum(per, keepdims=True) / N


def kernel(inputs, targets, global_inputs, global_targets, margin):
    del margin
    num_steps = M // BM
    t2 = targets.reshape(N, 1)
    gt2 = global_targets.reshape(1, M)
    out = pl.pallas_call(
        functools.partial(_body, num_steps=num_steps),
        grid=(num_steps,),
        in_specs=[
            pl.BlockSpec((N, D), lambda j: (0, 0)),
            pl.BlockSpec((N, 1), lambda j: (0, 0)),
            pl.BlockSpec((BM, D), lambda j: (j, 0)),
            pl.BlockSpec((1, BM), lambda j: (0, j)),
        ],
        out_specs=pl.BlockSpec((1, 1), lambda j: (0, 0)),
        out_shape=jax.ShapeDtypeStruct((1, 1), jnp.float32),
        scratch_shapes=[
            pltpu.VMEM((N, 1), jnp.float32),
            pltpu.VMEM((N, 1), jnp.float32),
            pltpu.VMEM((N, 1), jnp.float32),
            pltpu.VMEM((N, 1), jnp.float32),
        ],
        compiler_params=pltpu.CompilerParams(
            dimension_semantics=("arbitrary",),
        ),
    )(inputs, t2, global_inputs, gt2)
    return out.reshape(())
